# baseline (speedup 1.0000x reference)
import functools

import jax
import jax.numpy as jnp
from jax import lax
from jax.experimental import pallas as pl
from jax.experimental.pallas import tpu as pltpu
from jax.experimental.pallas import tpu_sc as plsc

NUM_HEADS = 16
N = 256
TABLE_WORDS = 961 * NUM_HEADS
NUM_WORKERS = 32
ROWS_PER_W = N // NUM_WORKERS
IDX_PER_W = ROWS_PER_W * N
BLK = N * NUM_HEADS
GROUPS = IDX_PER_W // 16


def _sc_gather(table_flat, idx_flat):
    mesh = plsc.VectorSubcoreMesh(core_axis_name="c", subcore_axis_name="s")

    @functools.partial(
        pl.kernel,
        mesh=mesh,
        out_type=jax.ShapeDtypeStruct((N * BLK,), jnp.float32),
        scratch_types=[
            pltpu.VMEM((TABLE_WORDS,), jnp.float32),
            pltpu.VMEM((IDX_PER_W,), jnp.int32),
            pltpu.VMEM((ROWS_PER_W * BLK,), jnp.float32),
            pltpu.SemaphoreType.DMA,
        ],
        compiler_params=pltpu.CompilerParams(
            use_tc_tiling_on_sc=False, needs_layout_passes=False),
    )
    def gather_kernel(table_hbm, idx_hbm, out_hbm, table_v, idx_v, blk_v, sem):
        wid = lax.axis_index("s") * 2 + lax.axis_index("c")
        with jax.named_scope("stage_in"):
            ct = pltpu.async_copy(table_hbm, table_v, sem)
            ci = pltpu.async_copy(idx_hbm.at[pl.ds(wid * IDX_PER_W, IDX_PER_W)],
                                  idx_v, sem)
            ct.wait()
            ci.wait()

        def body(r, _):
            idx16 = idx_v[pl.ds(r * 16, 16)]
            flat = idx16 * NUM_HEADS
            base = (r >> 4) * BLK + ((r >> 3) & 1) * 1024 + (r & 7) * 16
            for c in range(NUM_HEADS):
                vals = plsc.load_gather(table_v, [flat + c])
                blk_v[pl.ds(base + (c >> 3) * 2048 + (c & 7) * 128, 16)] = vals
            return ()

        with jax.named_scope("gather_loop"):
            lax.fori_loop(0, GROUPS, body, (), unroll=4)
        with jax.named_scope("write_out"):
            pltpu.sync_copy(
                blk_v,
                out_hbm.at[pl.ds(wid * ROWS_PER_W * BLK, ROWS_PER_W * BLK)])

    return gather_kernel(table_flat, idx_flat)


def kernel(relative_position_bias_table, relative_position_index):
    table_flat = relative_position_bias_table.reshape(-1)
    idx_flat = relative_position_index.astype(jnp.int32).reshape(-1)
    out = _sc_gather(table_flat, idx_flat)
    return (out.reshape(N, 2, 2, 8, 128)
               .transpose(0, 2, 4, 1, 3)
               .reshape(N, N, NUM_HEADS))


# speedup vs baseline: 1.2871x; 1.2871x over previous
import functools

import jax
import jax.numpy as jnp
from jax import lax
from jax.experimental import pallas as pl
from jax.experimental.pallas import tpu as pltpu
from jax.experimental.pallas import tpu_sc as plsc

NUM_HEADS = 16
N = 256
TABLE_WORDS = 961 * NUM_HEADS
NUM_WORKERS = 32
ROWS_PER_W = N // NUM_WORKERS
IDX_PER_W = ROWS_PER_W * N
BLK = N * NUM_HEADS
GROUPS = IDX_PER_W // 16


def _sc_gather(table_flat, idx_flat):
    mesh = plsc.VectorSubcoreMesh(core_axis_name="c", subcore_axis_name="s")

    @functools.partial(
        pl.kernel,
        mesh=mesh,
        out_type=jax.ShapeDtypeStruct((N * BLK,), jnp.float32),
        scratch_types=[
            pltpu.VMEM((TABLE_WORDS,), jnp.float32),
            pltpu.VMEM((IDX_PER_W,), jnp.int32),
            pltpu.VMEM((ROWS_PER_W * BLK,), jnp.float32),
            pltpu.SemaphoreType.DMA,
        ],
        compiler_params=pltpu.CompilerParams(
            use_tc_tiling_on_sc=False, needs_layout_passes=False),
    )
    def gather_kernel(table_hbm, idx_hbm, out_hbm, table_v, idx_v, blk_v, sem):
        wid = lax.axis_index("s") * 2 + lax.axis_index("c")
        with jax.named_scope("stage_in"):
            ct = pltpu.async_copy(table_hbm, table_v, sem)
            ci = pltpu.async_copy(idx_hbm.at[pl.ds(wid * IDX_PER_W, IDX_PER_W)],
                                  idx_v, sem)
            ct.wait()
            ci.wait()

        def body(r, _):
            idx16 = idx_v[pl.ds(r * 16, 16)]
            base = (r >> 4) * BLK + ((r >> 3) & 1) * 1024 + (r & 7) * 16
            for c in range(NUM_HEADS):
                vals = plsc.load_gather(table_v, [idx16 + c * 961])
                blk_v[pl.ds(base + (c >> 3) * 2048 + (c & 7) * 128, 16)] = vals
            return ()

        with jax.named_scope("gather_loop"):
            lax.fori_loop(0, GROUPS, body, (), unroll=4)
        with jax.named_scope("write_out"):
            pltpu.sync_copy(
                blk_v,
                out_hbm.at[pl.ds(wid * ROWS_PER_W * BLK, ROWS_PER_W * BLK)])

    return gather_kernel(table_flat, idx_flat)


def kernel(relative_position_bias_table, relative_position_index):
    table_flat = relative_position_bias_table.T.reshape(-1)
    idx_flat = relative_position_index.astype(jnp.int32).reshape(-1)
    out = _sc_gather(table_flat, idx_flat)
    return (out.reshape(N, 2, 2, 8, 128)
               .transpose(0, 2, 4, 1, 3)
               .reshape(N, N, NUM_HEADS))
